# baseline (speedup 1.0000x reference)
import functools

import jax
import jax.numpy as jnp
from jax import lax
from jax.experimental import pallas as pl
from jax.experimental.pallas import tpu as pltpu
from jax.experimental.pallas import tpu_sc as plsc

VOCAB = 1000000
D = 64
DP = 128
BATCH = 16384
FIELDS = 26
FLAT = BATCH * FIELDS

_info = plsc.get_sparse_core_info()
NC, NS = _info.num_cores, _info.num_subcores
NW = NC * NS
ROWS_PER_W = BATCH // NW
RPC = 4
CHUNK = RPC * FIELDS
NCHUNK = ROWS_PER_W // RPC


def _body(idx_hbm, tab_hbm, out_hbm, idx_v, rows_v, sem):
    wid = lax.axis_index("s") * NC + lax.axis_index("c")
    base = wid * NCHUNK * CHUNK
    pltpu.sync_copy(idx_hbm.at[wid], idx_v)

    def gather_start(j, slot):
        pltpu.async_copy(
            tab_hbm.at[idx_v.at[j].at[pl.ds(0, CHUNK)]],
            rows_v.at[slot],
            sem.at[slot],
        )

    def gather_wait(j, slot):
        pltpu.make_async_copy(
            tab_hbm.at[idx_v.at[j].at[pl.ds(0, CHUNK)]],
            rows_v.at[slot],
            sem.at[slot],
        ).wait()

    def write_out(j, slot):
        pltpu.sync_copy(
            rows_v.at[slot], out_hbm.at[pl.ds(base + j * CHUNK, CHUNK)]
        )

    gather_start(0, 0)

    def chunk(j, carry):
        slot = j & 1
        gather_wait(j, slot)
        gather_start(j + 1, 1 - slot)
        write_out(j, slot)
        return carry

    lax.fori_loop(0, NCHUNK - 1, chunk, 0)
    last = NCHUNK - 1
    gather_wait(last, last & 1)
    write_out(last, last & 1)


_lookup = functools.partial(
    pl.kernel,
    mesh=plsc.VectorSubcoreMesh(core_axis_name="c", subcore_axis_name="s"),
    out_type=jax.ShapeDtypeStruct((FLAT, DP), jnp.float32),
    scratch_types=[
        pltpu.VMEM((NCHUNK, 128), jnp.int32),
        pltpu.VMEM((2, CHUNK, DP), jnp.float32),
        pltpu.SemaphoreType.DMA((2,)),
    ],
)(_body)


def kernel(x, embedding_table):
    tab128 = jnp.pad(embedding_table, ((0, 0), (0, DP - D)))
    idx3 = x.reshape(-1).astype(jnp.int32).reshape(NW, NCHUNK, CHUNK)
    idx3 = jnp.pad(idx3, ((0, 0), (0, 0), (0, 128 - CHUNK)))
    out = _lookup(idx3, tab128)
    return out[:, :D].reshape(BATCH, FIELDS, D)


# speedup vs baseline: 1.2685x; 1.2685x over previous
import functools

import jax
import jax.numpy as jnp
from jax import lax
from jax.experimental import pallas as pl
from jax.experimental.pallas import tpu as pltpu
from jax.experimental.pallas import tpu_sc as plsc

VOCAB = 1000000
D = 64
BATCH = 16384
FIELDS = 26
FLAT = BATCH * FIELDS

_info = plsc.get_sparse_core_info()
NC, NS = _info.num_cores, _info.num_subcores
NW = NC * NS
ROWS_PER_W = BATCH // NW
RPC = 4
CHUNK = RPC * FIELDS
NCHUNK = ROWS_PER_W // RPC


def _body(idx_hbm, tab_hbm, out_hbm, idx_v, rows_v, sem):
    wid = lax.axis_index("s") * NC + lax.axis_index("c")
    row0 = wid * ROWS_PER_W
    pltpu.sync_copy(idx_hbm.at[wid], idx_v)

    def gather_start(j, slot):
        pltpu.async_copy(tab_hbm.at[idx_v.at[j]], rows_v.at[slot], sem.at[slot])

    def gather_wait(j, slot):
        pltpu.make_async_copy(
            tab_hbm.at[idx_v.at[j]], rows_v.at[slot], sem.at[slot]
        ).wait()

    def write_out(j, slot):
        for r in range(RPC):
            pltpu.sync_copy(
                rows_v.at[slot, pl.ds(r * FIELDS, FIELDS)],
                out_hbm.at[row0 + j * RPC + r, pl.ds(0, FIELDS), pl.ds(0, D)],
            )

    gather_start(0, 0)

    def chunk(j, carry):
        slot = j & 1
        gather_wait(j, slot)
        gather_start(j + 1, 1 - slot)
        write_out(j, slot)
        return carry

    lax.fori_loop(0, NCHUNK - 1, chunk, 0)
    last = NCHUNK - 1
    gather_wait(last, last & 1)
    write_out(last, last & 1)


_lookup = functools.partial(
    pl.kernel,
    mesh=plsc.VectorSubcoreMesh(core_axis_name="c", subcore_axis_name="s"),
    out_type=jax.ShapeDtypeStruct((BATCH, 32, 128), jnp.float32),
    scratch_types=[
        pltpu.VMEM((NCHUNK, CHUNK), jnp.int32),
        pltpu.VMEM((2, CHUNK, D), jnp.float32),
        pltpu.SemaphoreType.DMA((2,)),
    ],
    compiler_params=pltpu.CompilerParams(use_tc_tiling_on_sc=False),
)(_body)


def kernel(x, embedding_table):
    idx3 = x.reshape(-1).astype(jnp.int32).reshape(NW, NCHUNK, CHUNK)
    out = _lookup(idx3, embedding_table)
    return out[:, :FIELDS, :D]


# speedup vs baseline: 1.3435x; 1.0591x over previous
import functools

import jax
import jax.numpy as jnp
from jax import lax
from jax.experimental import pallas as pl
from jax.experimental.pallas import tpu as pltpu
from jax.experimental.pallas import tpu_sc as plsc

VOCAB = 1000000
D = 64
BATCH = 16384
FIELDS = 26
FLAT = BATCH * FIELDS

_info = plsc.get_sparse_core_info()
NC, NS = _info.num_cores, _info.num_subcores
NW = NC * NS
ROWS_PER_W = BATCH // NW
RPC = 8
CHUNK = RPC * FIELDS
NCHUNK = ROWS_PER_W // RPC


def _body(idx_hbm, tab_hbm, out_hbm, idx_v, rows_v, sem):
    wid = lax.axis_index("s") * NC + lax.axis_index("c")
    row0 = wid * ROWS_PER_W
    pltpu.sync_copy(idx_hbm.at[wid], idx_v)

    def gather_start(j, slot):
        pltpu.async_copy(tab_hbm.at[idx_v.at[j]], rows_v.at[slot], sem.at[slot])

    def gather_wait(j, slot):
        pltpu.make_async_copy(
            tab_hbm.at[idx_v.at[j]], rows_v.at[slot], sem.at[slot]
        ).wait()

    def write_out(j, slot):
        for r in range(RPC):
            pltpu.sync_copy(
                rows_v.at[slot, pl.ds(r * FIELDS, FIELDS)],
                out_hbm.at[row0 + j * RPC + r, pl.ds(0, FIELDS), pl.ds(0, D)],
            )

    gather_start(0, 0)

    def chunk(j, carry):
        slot = j & 1
        gather_wait(j, slot)
        gather_start(j + 1, 1 - slot)
        write_out(j, slot)
        return carry

    lax.fori_loop(0, NCHUNK - 1, chunk, 0)
    last = NCHUNK - 1
    gather_wait(last, last & 1)
    write_out(last, last & 1)


_lookup = functools.partial(
    pl.kernel,
    mesh=plsc.VectorSubcoreMesh(core_axis_name="c", subcore_axis_name="s"),
    out_type=jax.ShapeDtypeStruct((BATCH, 32, 128), jnp.float32),
    scratch_types=[
        pltpu.VMEM((NCHUNK, CHUNK), jnp.int32),
        pltpu.VMEM((2, CHUNK, D), jnp.float32),
        pltpu.SemaphoreType.DMA((2,)),
    ],
    compiler_params=pltpu.CompilerParams(use_tc_tiling_on_sc=False),
)(_body)


def kernel(x, embedding_table):
    idx3 = x.reshape(-1).astype(jnp.int32).reshape(NW, NCHUNK, CHUNK)
    out = _lookup(idx3, embedding_table)
    return out[:, :FIELDS, :D]


# speedup vs baseline: 1.3663x; 1.0170x over previous
import functools

import jax
import jax.numpy as jnp
from jax import lax
from jax.experimental import pallas as pl
from jax.experimental.pallas import tpu as pltpu
from jax.experimental.pallas import tpu_sc as plsc

VOCAB = 1000000
D = 64
BATCH = 16384
FIELDS = 26
FLAT = BATCH * FIELDS

_info = plsc.get_sparse_core_info()
NC, NS = _info.num_cores, _info.num_subcores
NW = NC * NS
ROWS_PER_W = BATCH // NW
RPC = 16
CHUNK = RPC * FIELDS
NCHUNK = ROWS_PER_W // RPC


def _body(idx_hbm, tab_hbm, out_hbm, idx_v, rows_v, sem):
    wid = lax.axis_index("s") * NC + lax.axis_index("c")
    row0 = wid * ROWS_PER_W
    pltpu.sync_copy(idx_hbm.at[wid], idx_v)

    def gather_start(j, slot):
        pltpu.async_copy(tab_hbm.at[idx_v.at[j]], rows_v.at[slot], sem.at[slot])

    def gather_wait(j, slot):
        pltpu.make_async_copy(
            tab_hbm.at[idx_v.at[j]], rows_v.at[slot], sem.at[slot]
        ).wait()

    def write_out(j, slot):
        for r in range(RPC):
            pltpu.sync_copy(
                rows_v.at[slot, pl.ds(r * FIELDS, FIELDS)],
                out_hbm.at[row0 + j * RPC + r, pl.ds(0, FIELDS), pl.ds(0, D)],
            )

    gather_start(0, 0)

    def chunk(j, carry):
        slot = j & 1
        gather_wait(j, slot)
        gather_start(j + 1, 1 - slot)
        write_out(j, slot)
        return carry

    lax.fori_loop(0, NCHUNK - 1, chunk, 0)
    last = NCHUNK - 1
    gather_wait(last, last & 1)
    write_out(last, last & 1)


_lookup = functools.partial(
    pl.kernel,
    mesh=plsc.VectorSubcoreMesh(core_axis_name="c", subcore_axis_name="s"),
    out_type=jax.ShapeDtypeStruct((BATCH, 32, 128), jnp.float32),
    scratch_types=[
        pltpu.VMEM((NCHUNK, CHUNK), jnp.int32),
        pltpu.VMEM((2, CHUNK, D), jnp.float32),
        pltpu.SemaphoreType.DMA((2,)),
    ],
    compiler_params=pltpu.CompilerParams(use_tc_tiling_on_sc=False),
)(_body)


def kernel(x, embedding_table):
    idx3 = x.reshape(-1).astype(jnp.int32).reshape(NW, NCHUNK, CHUNK)
    out = _lookup(idx3, embedding_table)
    return out[:, :FIELDS, :D]
